# baseline (speedup 1.0000x reference)
import jax
import jax.numpy as jnp
from jax import lax
from jax.experimental import pallas as pl
from jax.experimental.pallas import tpu as pltpu
from jax.experimental.pallas import tpu_sc as plsc

B = 4
S = 4096
D = 1024

NC = 2
NS = 16
NW = NC * NS

R = 8
ROWS_PER_W = S // NW
NCHUNKS = ROWS_PER_W // R
NSTEPS = NCHUNKS * B


NBUF = 10
PD = 6


def _sc_body(x_hbm, pe_hbm, o_hbm, pe_v, x_v, pe_sem, in_sem, out_sem):
    c = lax.axis_index("c")
    s = lax.axis_index("s")
    wid = s * NC + c
    base = wid * ROWS_PER_W

    pltpu.async_copy(pe_hbm.at[pl.ds(base, R)], pe_v.at[0], pe_sem.at[0])
    for t0 in range(PD):
        i0 = t0 // B
        b0 = t0 % B
        pltpu.async_copy(
            x_hbm.at[b0, pl.ds(base + i0 * R, R)], x_v.at[t0], in_sem.at[t0]
        )

    def step(t, _):
        i = t // B
        b = t % B
        slot = lax.rem(t, NBUF)
        row0 = base + i * R

        @pl.when(jnp.logical_and(b == 0, i + 1 < NCHUNKS))
        def _():
            pltpu.async_copy(
                pe_hbm.at[pl.ds(row0 + R, R)],
                pe_v.at[lax.rem(i + 1, 2)],
                pe_sem.at[lax.rem(i + 1, 2)],
            )

        @pl.when(t + PD < NSTEPS)
        def _():
            t2 = t + PD
            i2 = t2 // B
            b2 = t2 % B
            s2 = lax.rem(t2, NBUF)

            @pl.when(t >= NBUF - PD)
            def _():
                pltpu.make_async_copy(
                    x_v.at[s2], o_hbm.at[0, pl.ds(base, R)], out_sem.at[s2]
                ).wait()

            pltpu.async_copy(
                x_hbm.at[b2, pl.ds(base + i2 * R, R)], x_v.at[s2], in_sem.at[s2]
            )

        @pl.when(b == 0)
        def _():
            pltpu.make_async_copy(
                pe_hbm.at[pl.ds(row0, R)],
                pe_v.at[lax.rem(i, 2)],
                pe_sem.at[lax.rem(i, 2)],
            ).wait()

        pltpu.make_async_copy(
            x_hbm.at[b, pl.ds(row0, R)], x_v.at[slot], in_sem.at[slot]
        ).wait()

        pslot = lax.rem(i, 2)

        @plsc.parallel_loop(0, R * D, 16, unroll=16)
        def _add(j):
            r = j // D
            col = lax.rem(j, D)
            x_v[slot, r, pl.ds(col, 16)] = (
                x_v[slot, r, pl.ds(col, 16)] + pe_v[pslot, r, pl.ds(col, 16)]
            )

        pltpu.async_copy(x_v.at[slot], o_hbm.at[b, pl.ds(row0, R)], out_sem.at[slot])
        return 0

    lax.fori_loop(0, NSTEPS, step, 0)

    for k in range(NBUF):
        pltpu.make_async_copy(
            x_v.at[k], o_hbm.at[0, pl.ds(base, R)], out_sem.at[k]
        ).wait()


@jax.jit
def _sc_add(x, pe):
    mesh = plsc.VectorSubcoreMesh(core_axis_name="c", subcore_axis_name="s")
    f = pl.kernel(
        _sc_body,
        out_type=jax.ShapeDtypeStruct((B, S, D), jnp.float32),
        mesh=mesh,
        scratch_types=[
            pltpu.VMEM((2, R, D), jnp.float32),
            pltpu.VMEM((NBUF, R, D), jnp.float32),
            pltpu.SemaphoreType.DMA((2,)),
            pltpu.SemaphoreType.DMA((NBUF,)),
            pltpu.SemaphoreType.DMA((NBUF,)),
        ],
    )
    return f(x, pe)


def kernel(x, pos_embedding):
    return _sc_add(x, pos_embedding)
